# Initial kernel scaffold; baseline (speedup 1.0000x reference)
import jax
import jax.numpy as jnp
from jax.experimental import pallas as pl


def kernel(x, mask, inputP, W, a_src, a_dst, ln1_g, ln1_b, ln2_g, ln2_b):
    raise NotImplementedError("write your pallas kernel here")


# speedup vs baseline: 1.7688x; 1.7688x over previous
import functools

import jax
import jax.numpy as jnp
from jax.experimental import pallas as pl
from jax.experimental.pallas import tpu as pltpu

N = 4096
D_IN = 128
HIDDEN = 128
HEADS = 4
HEAD_DIM = HIDDEN // HEADS
ALPHA = 0.2
BR = 512
NEG = -1e9


def _ln(h, g, b, eps=1e-6):
    mu = jnp.mean(h, axis=-1, keepdims=True)
    var = jnp.mean((h - mu) ** 2, axis=-1, keepdims=True)
    return g * (h - mu) * jax.lax.rsqrt(var + eps) + b


def _block_kernel(x_ref, p_ref, w_ref, asrc_ref, adst_ref,
                  ln1g_ref, ln1b_ref, ln2g_ref, ln2b_ref,
                  out_ref, h_scr, fs_scr, fd_scr):
    i = pl.program_id(0)

    @pl.when(i == 0)
    def _prologue():
        xn = _ln(x_ref[...], ln1g_ref[0, :], ln1b_ref[0, :])
        for hd in range(HEADS):
            h_hd = jnp.dot(xn, w_ref[hd], preferred_element_type=jnp.float32)
            h_scr[:, hd * HEAD_DIM:(hd + 1) * HEAD_DIM] = h_hd
            fs = jnp.dot(h_hd, asrc_ref[hd, :][:, None],
                         preferred_element_type=jnp.float32)
            fd = jnp.dot(h_hd, adst_ref[hd, :][:, None],
                         preferred_element_type=jnp.float32)
            fs_scr[:, hd:hd + 1] = fs
            fd_scr[hd:hd + 1, :] = fd.reshape(1, N)

    adj = p_ref[...] > 0.9
    rows = pl.ds(i * BR, BR)
    outs = []
    for hd in range(HEADS):
        fs = fs_scr[rows, hd:hd + 1]
        fd = fd_scr[hd:hd + 1, :]
        e = fs + fd
        e = jnp.where(e >= 0, e, ALPHA * e)
        e = jnp.where(adj, e, NEG)
        m = jnp.max(e, axis=1, keepdims=True)
        p = jnp.exp(e - m)
        s = jnp.sum(p, axis=1, keepdims=True)
        attn = p / s
        outs.append(jnp.dot(attn, h_scr[:, hd * HEAD_DIM:(hd + 1) * HEAD_DIM],
                            preferred_element_type=jnp.float32))
    sub = jnp.concatenate(outs, axis=1)
    sub = jnp.where(sub > 0, sub, jnp.exp(sub) - 1.0)
    x2 = x_ref[rows, :] + sub
    out_ref[...] = _ln(x2, ln2g_ref[0, :], ln2b_ref[0, :])


@jax.jit
def kernel(x, mask, inputP, W, a_src, a_dst, ln1_g, ln1_b, ln2_g, ln2_b):
    del mask
    grid = (N // BR,)
    out = pl.pallas_call(
        _block_kernel,
        grid=grid,
        in_specs=[
            pl.BlockSpec((N, D_IN), lambda i: (0, 0)),
            pl.BlockSpec((BR, N), lambda i: (i, 0)),
            pl.BlockSpec((HEADS, D_IN, HEAD_DIM), lambda i: (0, 0, 0)),
            pl.BlockSpec((HEADS, HEAD_DIM), lambda i: (0, 0)),
            pl.BlockSpec((HEADS, HEAD_DIM), lambda i: (0, 0)),
            pl.BlockSpec((1, D_IN), lambda i: (0, 0)),
            pl.BlockSpec((1, D_IN), lambda i: (0, 0)),
            pl.BlockSpec((1, HIDDEN), lambda i: (0, 0)),
            pl.BlockSpec((1, HIDDEN), lambda i: (0, 0)),
        ],
        out_specs=pl.BlockSpec((BR, HIDDEN), lambda i: (i, 0)),
        out_shape=jax.ShapeDtypeStruct((N, HIDDEN), jnp.float32),
        scratch_shapes=[
            pltpu.VMEM((N, HIDDEN), jnp.float32),
            pltpu.VMEM((N, HEADS), jnp.float32),
            pltpu.VMEM((HEADS, N), jnp.float32),
        ],
    )(x, inputP, W, a_src, a_dst,
      ln1_g.reshape(1, D_IN), ln1_b.reshape(1, D_IN),
      ln2_g.reshape(1, HIDDEN), ln2_b.reshape(1, HIDDEN))
    return out


# speedup vs baseline: 1.9411x; 1.0974x over previous
import functools

import jax
import jax.numpy as jnp
from jax.experimental import pallas as pl
from jax.experimental.pallas import tpu as pltpu

N = 4096
D_IN = 128
HIDDEN = 128
HEADS = 4
HEAD_DIM = HIDDEN // HEADS
ALPHA = 0.2
BR = 256
NEG = -1e9


def _ln(h, g, b, eps=1e-6):
    mu = jnp.mean(h, axis=-1, keepdims=True)
    var = jnp.mean((h - mu) ** 2, axis=-1, keepdims=True)
    return g * (h - mu) * jax.lax.rsqrt(var + eps) + b


def _block_kernel(x_ref, p_ref, w_ref, asrc_ref, adst_ref,
                  ln1g_ref, ln1b_ref, ln2g_ref, ln2b_ref,
                  out_ref, h_scr, fs_scr, fd_scr):
    i = pl.program_id(0)

    @pl.when(i == 0)
    def _prologue():
        xn = _ln(x_ref[...], ln1g_ref[0, :], ln1b_ref[0, :])
        for hd in range(HEADS):
            h_hd = jnp.dot(xn, w_ref[hd], preferred_element_type=jnp.float32)
            h_scr[:, hd * HEAD_DIM:(hd + 1) * HEAD_DIM] = h_hd
            fs = jnp.dot(h_hd, asrc_ref[hd, :][:, None],
                         preferred_element_type=jnp.float32)
            fd = jnp.dot(h_hd, adst_ref[hd, :][:, None],
                         preferred_element_type=jnp.float32)
            fs_scr[:, hd:hd + 1] = fs
            fd_scr[hd:hd + 1, :] = fd.reshape(1, N)

    bias = jnp.where(p_ref[...] > 0.9, 0.0, NEG)
    rows = pl.ds(i * BR, BR)
    outs = []
    for hd in range(HEADS):
        fs = fs_scr[rows, hd:hd + 1]
        fd = fd_scr[hd:hd + 1, :]
        z = fs + fd
        e = jnp.maximum(z, ALPHA * z) + bias
        m = jnp.max(e, axis=1, keepdims=True)
        p = jnp.exp(e - m)
        s = jnp.sum(p, axis=1, keepdims=True)
        o = jnp.dot(p, h_scr[:, hd * HEAD_DIM:(hd + 1) * HEAD_DIM],
                    preferred_element_type=jnp.float32)
        outs.append(o / s)
    sub = jnp.concatenate(outs, axis=1)
    sub = jnp.where(sub > 0, sub, jnp.exp(sub) - 1.0)
    x2 = x_ref[rows, :] + sub
    out_ref[...] = _ln(x2, ln2g_ref[0, :], ln2b_ref[0, :])


@jax.jit
def kernel(x, mask, inputP, W, a_src, a_dst, ln1_g, ln1_b, ln2_g, ln2_b):
    del mask
    grid = (N // BR,)
    out = pl.pallas_call(
        _block_kernel,
        grid=grid,
        in_specs=[
            pl.BlockSpec((N, D_IN), lambda i: (0, 0)),
            pl.BlockSpec((BR, N), lambda i: (i, 0)),
            pl.BlockSpec((HEADS, D_IN, HEAD_DIM), lambda i: (0, 0, 0)),
            pl.BlockSpec((HEADS, HEAD_DIM), lambda i: (0, 0)),
            pl.BlockSpec((HEADS, HEAD_DIM), lambda i: (0, 0)),
            pl.BlockSpec((1, D_IN), lambda i: (0, 0)),
            pl.BlockSpec((1, D_IN), lambda i: (0, 0)),
            pl.BlockSpec((1, HIDDEN), lambda i: (0, 0)),
            pl.BlockSpec((1, HIDDEN), lambda i: (0, 0)),
        ],
        out_specs=pl.BlockSpec((BR, HIDDEN), lambda i: (i, 0)),
        out_shape=jax.ShapeDtypeStruct((N, HIDDEN), jnp.float32),
        scratch_shapes=[
            pltpu.VMEM((N, HIDDEN), jnp.float32),
            pltpu.VMEM((N, HEADS), jnp.float32),
            pltpu.VMEM((HEADS, N), jnp.float32),
        ],
    )(x, inputP, W, a_src, a_dst,
      ln1_g.reshape(1, D_IN), ln1_b.reshape(1, D_IN),
      ln2_g.reshape(1, HIDDEN), ln2_b.reshape(1, HIDDEN))
    return out


# speedup vs baseline: 2.8526x; 1.4695x over previous
import jax
import jax.numpy as jnp
from jax.experimental import pallas as pl
from jax.experimental.pallas import tpu as pltpu

N = 4096
D_IN = 128
HIDDEN = 128
HEADS = 4
HEAD_DIM = HIDDEN // HEADS
HW = 64
ALPHA = 0.2
BR = 256
NEG = -1e9


def _ln(h, g, b, eps=1e-6):
    mu = jnp.mean(h, axis=-1, keepdims=True)
    var = jnp.mean((h - mu) ** 2, axis=-1, keepdims=True)
    return g * (h - mu) * jax.lax.rsqrt(var + eps) + b


def _block_kernel(x_ref, p_ref, w_ref, asrc_ref, adst_ref,
                  ln1g_ref, ln1b_ref, ln2g_ref, ln2b_ref,
                  out_ref, h2_scr, fs_scr, fd_scr, gd_scr, fdm_scr, hm_scr):
    i = pl.program_id(0)

    @pl.when(i == 0)
    def _prologue():
        xn = _ln(x_ref[...], ln1g_ref[0, :], ln1b_ref[0, :])
        for hd in range(HEADS):
            h_hd = jnp.dot(xn, w_ref[hd], preferred_element_type=jnp.float32)
            h2_scr[:, hd * HW:(hd + 1) * HW] = jnp.concatenate(
                [h_hd, jnp.ones((N, 1), jnp.float32),
                 jnp.zeros((N, HW - HEAD_DIM - 1), jnp.float32)], axis=1)
            hm_scr[:, hd * HEAD_DIM:(hd + 1) * HEAD_DIM] = jnp.mean(
                h_hd, axis=0, keepdims=True)
            fs = jnp.dot(h_hd, asrc_ref[hd, :][:, None],
                         preferred_element_type=jnp.float32)
            fd = jnp.dot(h_hd, adst_ref[hd, :][:, None],
                         preferred_element_type=jnp.float32)
            fs_scr[:, hd:hd + 1] = fs
            fd_row = fd.reshape(1, N)
            fd_scr[hd:hd + 1, :] = fd_row
            gd_scr[hd:hd + 1, :] = ALPHA * fd_row
            fdm_scr[0:1, hd:hd + 1] = jnp.max(fd).reshape(1, 1)

    bias = jnp.where(p_ref[...] > 0.9, 0.0, NEG)
    rows = pl.ds(i * BR, BR)
    outs = []
    for hd in range(HEADS):
        fd = fd_scr[hd:hd + 1, :]
        gd = gd_scr[hd:hd + 1, :]
        fsb = fs_scr[rows, hd:hd + 1]
        zm = fsb + fdm_scr[0:1, hd:hd + 1]
        m = jnp.maximum(zm, ALPHA * zm)
        a1 = fsb - m
        a2 = ALPHA * fsb - m
        t = jnp.maximum(a1 + fd, a2 + gd) + bias
        p = jnp.exp(t)
        r = jnp.dot(p, h2_scr[:, hd * HW:(hd + 1) * HW],
                    preferred_element_type=jnp.float32)
        o = r[:, :HEAD_DIM]
        s = r[:, HEAD_DIM:HEAD_DIM + 1]
        outs.append(jnp.where(s > 0, o / s,
                              hm_scr[:, hd * HEAD_DIM:(hd + 1) * HEAD_DIM]))
    sub = jnp.concatenate(outs, axis=1)
    sub = jnp.where(sub > 0, sub, jnp.exp(sub) - 1.0)
    x2 = x_ref[rows, :] + sub
    out_ref[...] = _ln(x2, ln2g_ref[0, :], ln2b_ref[0, :])


@jax.jit
def kernel(x, mask, inputP, W, a_src, a_dst, ln1_g, ln1_b, ln2_g, ln2_b):
    del mask
    grid = (N // BR,)
    out = pl.pallas_call(
        _block_kernel,
        grid=grid,
        in_specs=[
            pl.BlockSpec((N, D_IN), lambda i: (0, 0)),
            pl.BlockSpec((BR, N), lambda i: (i, 0)),
            pl.BlockSpec((HEADS, D_IN, HEAD_DIM), lambda i: (0, 0, 0)),
            pl.BlockSpec((HEADS, HEAD_DIM), lambda i: (0, 0)),
            pl.BlockSpec((HEADS, HEAD_DIM), lambda i: (0, 0)),
            pl.BlockSpec((1, D_IN), lambda i: (0, 0)),
            pl.BlockSpec((1, D_IN), lambda i: (0, 0)),
            pl.BlockSpec((1, HIDDEN), lambda i: (0, 0)),
            pl.BlockSpec((1, HIDDEN), lambda i: (0, 0)),
        ],
        compiler_params=pltpu.CompilerParams(
            vmem_limit_bytes=100 * 1024 * 1024),
        out_specs=pl.BlockSpec((BR, HIDDEN), lambda i: (i, 0)),
        out_shape=jax.ShapeDtypeStruct((N, HIDDEN), jnp.float32),
        scratch_shapes=[
            pltpu.VMEM((N, HEADS * HW), jnp.float32),
            pltpu.VMEM((N, HEADS), jnp.float32),
            pltpu.VMEM((HEADS, N), jnp.float32),
            pltpu.VMEM((HEADS, N), jnp.float32),
            pltpu.VMEM((1, HEADS), jnp.float32),
            pltpu.VMEM((1, HIDDEN), jnp.float32),
        ],
    )(x, inputP, W, a_src, a_dst,
      ln1_g.reshape(1, D_IN), ln1_b.reshape(1, D_IN),
      ln2_g.reshape(1, HIDDEN), ln2_b.reshape(1, HIDDEN))
    return out


# speedup vs baseline: 3.7978x; 1.3313x over previous
import jax
import jax.numpy as jnp
from jax.experimental import pallas as pl
from jax.experimental.pallas import tpu as pltpu

N = 4096
D_IN = 128
HIDDEN = 128
HEADS = 4
HEAD_DIM = HIDDEN // HEADS
HW = 64
ALPHA = 0.2
BR = 256
NEG = -1e9


def _ln(h, g, b, eps=1e-6):
    mu = jnp.mean(h, axis=-1, keepdims=True)
    var = jnp.mean((h - mu) ** 2, axis=-1, keepdims=True)
    return g * (h - mu) * jax.lax.rsqrt(var + eps) + b


def _block_kernel(x_ref, p_ref, w_ref, csrc_ref, cdst_ref,
                  ln1g_ref, ln1b_ref, ln2g_ref, ln2b_ref,
                  out_ref, h2_scr, fs_scr, v1_scr, v2_scr, fdm_scr, hm_scr):
    i = pl.program_id(0)

    @pl.when(i == 0)
    def _prologue():
        xn = _ln(x_ref[...], ln1g_ref[0, :], ln1b_ref[0, :])
        fs_scr[...] = jnp.dot(xn, csrc_ref[...],
                              preferred_element_type=jnp.float32)
        fd = jax.lax.dot_general(
            cdst_ref[...], xn, (((1,), (1,)), ((), ())),
            preferred_element_type=jnp.float32)
        fdm = jnp.max(fd, axis=1, keepdims=True)
        fdm_scr[...] = fdm
        fd0 = fd - fdm
        v1_scr[...] = jnp.exp(fd0)
        v2_scr[...] = jnp.exp(ALPHA * fd0)
        for hd in range(HEADS):
            h_hd = jnp.dot(xn, w_ref[hd], preferred_element_type=jnp.float32)
            h2_scr[:, hd * HW:(hd + 1) * HW] = jnp.concatenate(
                [h_hd, jnp.ones((N, 1), jnp.float32),
                 jnp.zeros((N, HW - HEAD_DIM - 1), jnp.float32)], axis=1)
            hm_scr[:, hd * HEAD_DIM:(hd + 1) * HEAD_DIM] = jnp.mean(
                h_hd, axis=0, keepdims=True)

    a01 = jnp.where(p_ref[...] > 0.9, 1.0, 0.0)
    rows = pl.ds(i * BR, BR)
    outs = []
    for hd in range(HEADS):
        fsb = fs_scr[rows, hd:hd + 1]
        zm = fsb + fdm_scr[hd:hd + 1, 0:1]
        m = jnp.maximum(zm, ALPHA * zm)
        u1 = jnp.exp(zm - m)
        u2 = jnp.exp(ALPHA * zm - m)
        p = jnp.maximum(u1 * v1_scr[hd:hd + 1, :],
                        u2 * v2_scr[hd:hd + 1, :]) * a01
        r = jnp.dot(p, h2_scr[:, hd * HW:(hd + 1) * HW],
                    preferred_element_type=jnp.float32)
        o = r[:, :HEAD_DIM]
        s = r[:, HEAD_DIM:HEAD_DIM + 1]
        outs.append(jnp.where(s > 0, o / s,
                              hm_scr[:, hd * HEAD_DIM:(hd + 1) * HEAD_DIM]))
    sub = jnp.concatenate(outs, axis=1)
    sub = jnp.where(sub > 0, sub, jnp.exp(sub) - 1.0)
    x2 = x_ref[rows, :] + sub
    out_ref[...] = _ln(x2, ln2g_ref[0, :], ln2b_ref[0, :])


@jax.jit
def kernel(x, mask, inputP, W, a_src, a_dst, ln1_g, ln1_b, ln2_g, ln2_b):
    del mask
    csrc = jnp.einsum('hdk,hk->dh', W, a_src)
    cdst = jnp.einsum('hdk,hk->hd', W, a_dst)
    grid = (N // BR,)
    out = pl.pallas_call(
        _block_kernel,
        grid=grid,
        in_specs=[
            pl.BlockSpec((N, D_IN), lambda i: (0, 0)),
            pl.BlockSpec((BR, N), lambda i: (i, 0)),
            pl.BlockSpec((HEADS, D_IN, HEAD_DIM), lambda i: (0, 0, 0)),
            pl.BlockSpec((D_IN, HEADS), lambda i: (0, 0)),
            pl.BlockSpec((HEADS, D_IN), lambda i: (0, 0)),
            pl.BlockSpec((1, D_IN), lambda i: (0, 0)),
            pl.BlockSpec((1, D_IN), lambda i: (0, 0)),
            pl.BlockSpec((1, HIDDEN), lambda i: (0, 0)),
            pl.BlockSpec((1, HIDDEN), lambda i: (0, 0)),
        ],
        compiler_params=pltpu.CompilerParams(
            vmem_limit_bytes=100 * 1024 * 1024),
        out_specs=pl.BlockSpec((BR, HIDDEN), lambda i: (i, 0)),
        out_shape=jax.ShapeDtypeStruct((N, HIDDEN), jnp.float32),
        scratch_shapes=[
            pltpu.VMEM((N, HEADS * HW), jnp.float32),
            pltpu.VMEM((N, HEADS), jnp.float32),
            pltpu.VMEM((HEADS, N), jnp.float32),
            pltpu.VMEM((HEADS, N), jnp.float32),
            pltpu.VMEM((HEADS, 1), jnp.float32),
            pltpu.VMEM((1, HIDDEN), jnp.float32),
        ],
    )(x, inputP, W, csrc, cdst,
      ln1_g.reshape(1, D_IN), ln1_b.reshape(1, D_IN),
      ln2_g.reshape(1, HIDDEN), ln2_b.reshape(1, HIDDEN))
    return out


# speedup vs baseline: 4.3323x; 1.1407x over previous
import jax
import jax.numpy as jnp
from jax.experimental import pallas as pl
from jax.experimental.pallas import tpu as pltpu

N = 4096
D_IN = 128
HIDDEN = 128
HEADS = 4
HEAD_DIM = HIDDEN // HEADS
HW = 64
ALPHA = 0.2
BR = 256
NEG = -1e9


def _ln(h, g, b, eps=1e-6):
    mu = jnp.mean(h, axis=-1, keepdims=True)
    var = jnp.mean((h - mu) ** 2, axis=-1, keepdims=True)
    return g * (h - mu) * jax.lax.rsqrt(var + eps) + b


def _block_kernel(x_ref, p_ref, w_ref, csrc_ref, cdst_ref,
                  ln1g_ref, ln1b_ref, ln2g_ref, ln2b_ref,
                  out_ref, h2_scr, fs_scr, v1_scr, v2_scr, fdm_scr, hm_scr):
    i = pl.program_id(0)

    @pl.when(i == 0)
    def _prologue():
        xn = _ln(x_ref[...], ln1g_ref[0, :], ln1b_ref[0, :])
        fs_scr[...] = jnp.dot(xn, csrc_ref[...],
                              preferred_element_type=jnp.float32)
        fd = jax.lax.dot_general(
            cdst_ref[...], xn, (((1,), (1,)), ((), ())),
            preferred_element_type=jnp.float32)
        fdm = jnp.max(fd, axis=1, keepdims=True)
        fdm_scr[...] = fdm
        fd0 = fd - fdm
        v1_scr[...] = jnp.exp(fd0).astype(jnp.bfloat16)
        v2_scr[...] = jnp.exp(ALPHA * fd0).astype(jnp.bfloat16)
        for hd in range(HEADS):
            h_hd = jnp.dot(xn, w_ref[hd], preferred_element_type=jnp.float32)
            h2_scr[:, hd * HW:(hd + 1) * HW] = jnp.concatenate(
                [h_hd, jnp.ones((N, 1), jnp.float32),
                 jnp.zeros((N, HW - HEAD_DIM - 1), jnp.float32)],
                axis=1).astype(jnp.bfloat16)
            hm_scr[:, hd * HEAD_DIM:(hd + 1) * HEAD_DIM] = jnp.mean(
                h_hd, axis=0, keepdims=True)

    a01 = jnp.where(p_ref[...] > 0.9, 1.0, 0.0).astype(jnp.bfloat16)
    rows = pl.ds(i * BR, BR)
    outs = []
    for hd in range(HEADS):
        fsb = fs_scr[rows, hd:hd + 1]
        zm = fsb + fdm_scr[hd:hd + 1, 0:1]
        m = jnp.maximum(zm, ALPHA * zm)
        u1 = jnp.exp(zm - m).astype(jnp.bfloat16)
        u2 = jnp.exp(ALPHA * zm - m).astype(jnp.bfloat16)
        p = jnp.maximum(u1 * v1_scr[hd:hd + 1, :],
                        u2 * v2_scr[hd:hd + 1, :]) * a01
        r = jnp.dot(p, h2_scr[:, hd * HW:(hd + 1) * HW],
                    preferred_element_type=jnp.float32)
        o = r[:, :HEAD_DIM]
        s = r[:, HEAD_DIM:HEAD_DIM + 1]
        outs.append(jnp.where(s > 0, o / s,
                              hm_scr[:, hd * HEAD_DIM:(hd + 1) * HEAD_DIM]))
    sub = jnp.concatenate(outs, axis=1)
    sub = jnp.where(sub > 0, sub, jnp.exp(sub) - 1.0)
    x2 = x_ref[rows, :] + sub
    out_ref[...] = _ln(x2, ln2g_ref[0, :], ln2b_ref[0, :])


@jax.jit
def kernel(x, mask, inputP, W, a_src, a_dst, ln1_g, ln1_b, ln2_g, ln2_b):
    del mask
    csrc = jnp.einsum('hdk,hk->dh', W, a_src)
    cdst = jnp.einsum('hdk,hk->hd', W, a_dst)
    grid = (N // BR,)
    out = pl.pallas_call(
        _block_kernel,
        grid=grid,
        in_specs=[
            pl.BlockSpec((N, D_IN), lambda i: (0, 0)),
            pl.BlockSpec((BR, N), lambda i: (i, 0)),
            pl.BlockSpec((HEADS, D_IN, HEAD_DIM), lambda i: (0, 0, 0)),
            pl.BlockSpec((D_IN, HEADS), lambda i: (0, 0)),
            pl.BlockSpec((HEADS, D_IN), lambda i: (0, 0)),
            pl.BlockSpec((1, D_IN), lambda i: (0, 0)),
            pl.BlockSpec((1, D_IN), lambda i: (0, 0)),
            pl.BlockSpec((1, HIDDEN), lambda i: (0, 0)),
            pl.BlockSpec((1, HIDDEN), lambda i: (0, 0)),
        ],
        compiler_params=pltpu.CompilerParams(
            vmem_limit_bytes=100 * 1024 * 1024),
        out_specs=pl.BlockSpec((BR, HIDDEN), lambda i: (i, 0)),
        out_shape=jax.ShapeDtypeStruct((N, HIDDEN), jnp.float32),
        scratch_shapes=[
            pltpu.VMEM((N, HEADS * HW), jnp.bfloat16),
            pltpu.VMEM((N, HEADS), jnp.float32),
            pltpu.VMEM((HEADS, N), jnp.bfloat16),
            pltpu.VMEM((HEADS, N), jnp.bfloat16),
            pltpu.VMEM((HEADS, 1), jnp.float32),
            pltpu.VMEM((1, HIDDEN), jnp.float32),
        ],
    )(x, inputP, W, csrc, cdst,
      ln1_g.reshape(1, D_IN), ln1_b.reshape(1, D_IN),
      ln2_g.reshape(1, HIDDEN), ln2_b.reshape(1, HIDDEN))
    return out


# speedup vs baseline: 4.6596x; 1.0756x over previous
import jax
import jax.numpy as jnp
from jax.experimental import pallas as pl
from jax.experimental.pallas import tpu as pltpu

N = 4096
D_IN = 128
HIDDEN = 128
HEADS = 4
HEAD_DIM = HIDDEN // HEADS
HW = 64
ALPHA = 0.2
BR = 512
NEG = -1e9


def _ln(h, g, b, eps=1e-6):
    mu = jnp.mean(h, axis=-1, keepdims=True)
    var = jnp.mean((h - mu) ** 2, axis=-1, keepdims=True)
    return g * (h - mu) * jax.lax.rsqrt(var + eps) + b


def _block_kernel(x_ref, p_ref, w_ref, csrc_ref, cdst_ref,
                  ln1g_ref, ln1b_ref, ln2g_ref, ln2b_ref,
                  out_ref, h2_scr, fs_scr, v1_scr, v2_scr, fdm_scr, hm_scr):
    i = pl.program_id(0)

    @pl.when(i == 0)
    def _prologue():
        xn = _ln(x_ref[...], ln1g_ref[0, :], ln1b_ref[0, :])
        fs_scr[...] = jnp.dot(xn, csrc_ref[...],
                              preferred_element_type=jnp.float32)
        fd = jax.lax.dot_general(
            cdst_ref[...], xn, (((1,), (1,)), ((), ())),
            preferred_element_type=jnp.float32)
        fdm = jnp.max(fd, axis=1, keepdims=True)
        fdm_scr[...] = fdm
        fd0 = fd - fdm
        v1_scr[...] = jnp.exp(fd0).astype(jnp.bfloat16)
        v2_scr[...] = jnp.exp(ALPHA * fd0).astype(jnp.bfloat16)
        for hd in range(HEADS):
            h_hd = jnp.dot(xn, w_ref[hd], preferred_element_type=jnp.float32)
            h2_scr[:, hd * HW:(hd + 1) * HW] = jnp.concatenate(
                [h_hd, jnp.ones((N, 1), jnp.float32),
                 jnp.zeros((N, HW - HEAD_DIM - 1), jnp.float32)],
                axis=1).astype(jnp.bfloat16)
            hm_scr[:, hd * HEAD_DIM:(hd + 1) * HEAD_DIM] = jnp.mean(
                h_hd, axis=0, keepdims=True)

    a01 = jnp.where(p_ref[...] > 0.9, 1.0, 0.0).astype(jnp.bfloat16)
    rows = pl.ds(i * BR, BR)
    outs = []
    for hd in range(HEADS):
        fsb = fs_scr[rows, hd:hd + 1]
        zm = fsb + fdm_scr[hd:hd + 1, 0:1]
        m = jnp.maximum(zm, ALPHA * zm)
        u1 = jnp.exp(zm - m).astype(jnp.bfloat16)
        u2 = jnp.exp(ALPHA * zm - m).astype(jnp.bfloat16)
        p = jnp.maximum(u1 * v1_scr[hd:hd + 1, :],
                        u2 * v2_scr[hd:hd + 1, :]) * a01
        r = jnp.dot(p, h2_scr[:, hd * HW:(hd + 1) * HW],
                    preferred_element_type=jnp.float32)
        o = r[:, :HEAD_DIM]
        s = r[:, HEAD_DIM:HEAD_DIM + 1]
        outs.append(jnp.where(s > 0, o / s,
                              hm_scr[:, hd * HEAD_DIM:(hd + 1) * HEAD_DIM]))
    sub = jnp.concatenate(outs, axis=1)
    sub = jnp.where(sub > 0, sub, jnp.exp(sub) - 1.0)
    x2 = x_ref[rows, :] + sub
    out_ref[...] = _ln(x2, ln2g_ref[0, :], ln2b_ref[0, :])


@jax.jit
def kernel(x, mask, inputP, W, a_src, a_dst, ln1_g, ln1_b, ln2_g, ln2_b):
    del mask
    csrc = jnp.einsum('hdk,hk->dh', W, a_src)
    cdst = jnp.einsum('hdk,hk->hd', W, a_dst)
    grid = (N // BR,)
    out = pl.pallas_call(
        _block_kernel,
        grid=grid,
        in_specs=[
            pl.BlockSpec((N, D_IN), lambda i: (0, 0)),
            pl.BlockSpec((BR, N), lambda i: (i, 0)),
            pl.BlockSpec((HEADS, D_IN, HEAD_DIM), lambda i: (0, 0, 0)),
            pl.BlockSpec((D_IN, HEADS), lambda i: (0, 0)),
            pl.BlockSpec((HEADS, D_IN), lambda i: (0, 0)),
            pl.BlockSpec((1, D_IN), lambda i: (0, 0)),
            pl.BlockSpec((1, D_IN), lambda i: (0, 0)),
            pl.BlockSpec((1, HIDDEN), lambda i: (0, 0)),
            pl.BlockSpec((1, HIDDEN), lambda i: (0, 0)),
        ],
        compiler_params=pltpu.CompilerParams(
            vmem_limit_bytes=100 * 1024 * 1024),
        out_specs=pl.BlockSpec((BR, HIDDEN), lambda i: (i, 0)),
        out_shape=jax.ShapeDtypeStruct((N, HIDDEN), jnp.float32),
        scratch_shapes=[
            pltpu.VMEM((N, HEADS * HW), jnp.bfloat16),
            pltpu.VMEM((N, HEADS), jnp.float32),
            pltpu.VMEM((HEADS, N), jnp.bfloat16),
            pltpu.VMEM((HEADS, N), jnp.bfloat16),
            pltpu.VMEM((HEADS, 1), jnp.float32),
            pltpu.VMEM((1, HIDDEN), jnp.float32),
        ],
    )(x, inputP, W, csrc, cdst,
      ln1_g.reshape(1, D_IN), ln1_b.reshape(1, D_IN),
      ln2_g.reshape(1, HIDDEN), ln2_b.reshape(1, HIDDEN))
    return out
